# baseline (speedup 1.0000x reference)
import functools

import jax
import jax.numpy as jnp
from jax import lax
from jax.experimental import pallas as pl
from jax.experimental.pallas import tpu as pltpu
from jax.experimental.pallas import tpu_sc as plsc

K = 16
B = 8
N = 2048
C = 64
C2 = 2 * C
RT = 256
NT = N // RT
BN = B * N
ROWS = BN * K
HROWS = ROWS // 2
M_ELEMS = float(ROWS)
NEG = -3.0e38
EPS = 1e-5

NC = 2
NS = 16
NW = NC * NS
ROWS_W = ROWS // NW
SUB = 128
NSUB = ROWS_W // SUB
CHROWS = 512
NCH = ROWS_W // CHROWS
SUB_CH = CHROWS // SUB


def _knn_pre_body(ptsc_ref, ptsr_ref, ft_ref, wdt_ref, wbt_ref,
                  idx_ref, a_ref, g_ref):
    b = pl.program_id(0)
    ptsc = ptsc_ref[0]
    ptsr = ptsr_ref[0]
    xx_c = jnp.sum(ptsc * ptsc, axis=0, keepdims=True)
    xx_r = jnp.sum(ptsr * ptsr, axis=1, keepdims=True)
    ptscb = ptsc.astype(jnp.bfloat16).astype(jnp.float32)
    ptsrb = ptsr.astype(jnp.bfloat16).astype(jnp.float32)
    acc = (ptsrb[:, 0:1] * ptscb[0:1, :]
           + ptsrb[:, 1:2] * ptscb[1:2, :]
           + ptsrb[:, 2:3] * ptscb[2:3, :])
    d = (-xx_r - (-2.0 * acc)) - xx_c
    colv = lax.broadcasted_iota(jnp.int32, (1, N), 1)
    hi = (colv // 16).astype(jnp.float32)
    lo = (colv % 16).astype(jnp.float32)
    hilo = jnp.concatenate([hi, lo], axis=0)
    picked = []
    for j in range(K + 1):
        m = jnp.max(d, axis=1, keepdims=True)
        eq = d == m
        if j > 0:
            eqf = jnp.where(eq, 1.0, 0.0)
            hl = jax.lax.dot_general(
                eqf, hilo, (((1,), (1,)), ((), ())),
                preferred_element_type=jnp.float32)
            picked.append(hl[:, 0:1] * 16.0 + hl[:, 1:2])
        if j < K:
            d = jnp.where(eq, NEG, d)
    sel = jnp.minimum(jnp.concatenate(picked, axis=1), float(N - 1))
    idx_ref[0] = sel.astype(jnp.int32) + b * N
    ft = ft_ref[0]
    a_ref[0] = jnp.dot(ft, wdt_ref[...], precision=lax.Precision.HIGHEST,
                       preferred_element_type=jnp.float32)
    g_ref[0] = jnp.dot(ft, wbt_ref[...], precision=lax.Precision.HIGHEST,
                       preferred_element_type=jnp.float32)


_knn_pre = pl.pallas_call(
    _knn_pre_body,
    grid=(B, NT),
    in_specs=[
        pl.BlockSpec((1, 3, N), lambda b, t: (b, 0, 0)),
        pl.BlockSpec((1, RT, 3), lambda b, t: (b, t, 0)),
        pl.BlockSpec((1, RT, C), lambda b, t: (b, t, 0)),
        pl.BlockSpec((C, C), lambda b, t: (0, 0)),
        pl.BlockSpec((C, C), lambda b, t: (0, 0)),
    ],
    out_specs=[
        pl.BlockSpec((1, RT, K), lambda b, t: (b, t, 0)),
        pl.BlockSpec((1, RT, C), lambda b, t: (b, t, 0)),
        pl.BlockSpec((1, RT, C), lambda b, t: (b, t, 0)),
    ],
    out_shape=[
        jax.ShapeDtypeStruct((B, N, K), jnp.int32),
        jax.ShapeDtypeStruct((B, N, C), jnp.float32),
        jax.ShapeDtypeStruct((B, N, C), jnp.float32),
    ],
)


def _sc_gather_body(table_hbm, idx_hbm, out_hbm, idx_v, rows_a, rows_b, sem,
                    semo):
    wid = lax.axis_index("s") * NC + lax.axis_index("c")
    pltpu.sync_copy(idx_hbm.at[wid], idx_v)
    bufs = (rows_a, rows_b)
    pending = [None, None]
    for ch in range(NCH):
        buf = bufs[ch % 2]
        if pending[ch % 2] is not None:
            pending[ch % 2].wait()
        descs = []
        for j in range(SUB_CH):
            descs.append(pltpu.async_copy(
                table_hbm.at[idx_v.at[ch * SUB_CH + j]],
                buf.at[pl.ds(j * SUB, SUB)], sem))
        for dsc in descs:
            dsc.wait()
        pending[ch % 2] = pltpu.async_copy(
            buf, out_hbm.at[pl.ds(wid * ROWS_W + ch * CHROWS, CHROWS)], semo)
    pending[0].wait()
    pending[1].wait()


@functools.cache
def _make_sc_gather():
    return functools.partial(
        pl.kernel,
        out_type=jax.ShapeDtypeStruct((ROWS, C), jnp.float32),
        mesh=plsc.VectorSubcoreMesh(core_axis_name="c", subcore_axis_name="s",
                                    num_cores=NC, num_subcores=NS),
        scratch_types=[
            pltpu.VMEM((NSUB, SUB), jnp.int32),
            pltpu.VMEM((CHROWS, C), jnp.float32),
            pltpu.VMEM((CHROWS, C), jnp.float32),
            pltpu.SemaphoreType.DMA,
            pltpu.SemaphoreType.DMA,
        ],
        compiler_params=pltpu.CompilerParams(use_tc_tiling_on_sc=False),
    )(_sc_gather_body)


HROWB = 4096
NODB = HROWB // 8
NSTEP = HROWS // HROWB


def _coeffs(st, g_ref, b_ref):
    s = st[0:1, :C] + st[0:1, C:]
    ss = st[1:2, :C] + st[1:2, C:]
    mean = s / M_ELEMS
    var = ss / M_ELEMS - mean * mean
    scale = g_ref[...] * lax.rsqrt(var + EPS)
    shift = b_ref[...] - mean * scale
    scale2 = jnp.concatenate([scale, scale], axis=1)
    shift2 = jnp.concatenate([shift, shift], axis=1)
    return scale2, shift2


def _acc(st_ref, x):
    i = pl.program_id(1)

    @pl.when(i == 0)
    def _():
        st_ref[...] = jnp.zeros_like(st_ref)

    s = jnp.sum(x, axis=0, keepdims=True)
    ss = jnp.sum(x * x, axis=0, keepdims=True)
    st_ref[0:1, :] = st_ref[0:1, :] + s
    st_ref[1:2, :] = st_ref[1:2, :] + ss


def _mlp_body(gg_ref, a_ref, w1_ref, w2_ref, g0_ref, b0_ref, g1_ref,
              b1_ref, g2_ref, b2_ref, ft_ref, o_ref, st1, st2, st3):
    ph = pl.program_id(0)

    def x1_block():
        a2 = jnp.concatenate([a_ref[...], a_ref[...]], axis=1)
        return (gg_ref[...].reshape(NODB, 8, C2)
                + a2[:, None, :]).reshape(HROWB, C2)

    def x2_block():
        scale, shift = _coeffs(st1[...], g0_ref, b0_ref)
        y = jnp.maximum(x1_block() * scale + shift, 0.0).astype(jnp.bfloat16)
        return jnp.dot(y, w1_ref[...], preferred_element_type=jnp.float32)

    def x3_block():
        scale, shift = _coeffs(st2[...], g1_ref, b1_ref)
        y = jnp.maximum(x2_block() * scale + shift, 0.0).astype(jnp.bfloat16)
        return jnp.dot(y, w2_ref[...], preferred_element_type=jnp.float32)

    def ph0():
        _acc(st1, x1_block())

    def ph1():
        _acc(st2, x2_block())

    def ph2():
        _acc(st3, x3_block())

    def ph3():
        scale, shift = _coeffs(st3[...], g2_ref, b2_ref)
        y = jnp.maximum(x3_block() * scale + shift, 0.0)
        acc = jnp.sum(y.reshape(NODB, 8, C2), axis=1)
        fts = (acc[:, :C] + acc[:, C:]) * (1.0 / K)
        o_ref[...] = jnp.maximum(ft_ref[...] + fts, 0.0)

    lax.switch(ph, (ph0, ph1, ph2, ph3))


def _small(ix=None):
    if ix is None:
        return pl.BlockSpec((1, C), lambda p, i: (0, 0))
    return ix


_mlp = pl.pallas_call(
    _mlp_body,
    grid=(4, NSTEP),
    in_specs=[
        pl.BlockSpec((HROWB, C2), lambda p, i: (i, 0)),
        pl.BlockSpec((NODB, C), lambda p, i: (i, 0)),
        pl.BlockSpec((C2, C2), lambda p, i: (0, 0)),
        pl.BlockSpec((C2, C2), lambda p, i: (0, 0)),
        _small(), _small(), _small(), _small(), _small(), _small(),
        pl.BlockSpec((NODB, C), lambda p, i: (i, 0)),
    ],
    out_specs=pl.BlockSpec((NODB, C), lambda p, i: (i, 0)),
    out_shape=jax.ShapeDtypeStruct((BN, C), jnp.float32),
    scratch_shapes=[
        pltpu.VMEM((8, C2), jnp.float32),
        pltpu.VMEM((8, C2), jnp.float32),
        pltpu.VMEM((8, C2), jnp.float32),
    ],
)


def _dup_blockdiag(w):
    wt = jnp.transpose(w).astype(jnp.bfloat16)
    z = jnp.zeros((C, C), jnp.bfloat16)
    return jnp.concatenate(
        [jnp.concatenate([wt, z], axis=1),
         jnp.concatenate([z, wt], axis=1)], axis=0)


def kernel(points, features, W0, W1, W2, g0, b0, g1, b1, g2, b2):
    ft = jnp.transpose(features, (0, 2, 1))
    pt = jnp.transpose(points, (0, 2, 1))
    wdt = jnp.transpose(W0[:, :C] - W0[:, C:])
    wbt = jnp.transpose(W0[:, C:])
    idx, a_nodes, g_nodes = _knn_pre(points, pt, ft, wdt, wbt)
    gg = _make_sc_gather()(g_nodes.reshape(BN, C), idx.reshape(NW, NSUB, SUB))
    out = _mlp(gg.reshape(HROWS, C2), a_nodes.reshape(BN, C),
               _dup_blockdiag(W1), _dup_blockdiag(W2),
               g0.reshape(1, C), b0.reshape(1, C),
               g1.reshape(1, C), b1.reshape(1, C),
               g2.reshape(1, C), b2.reshape(1, C), ft.reshape(BN, C))
    return jnp.transpose(out.reshape(B, N, C), (0, 2, 1))


# speedup vs baseline: 1.0125x; 1.0125x over previous
import functools

import jax
import jax.numpy as jnp
from jax import lax
from jax.experimental import pallas as pl
from jax.experimental.pallas import tpu as pltpu
from jax.experimental.pallas import tpu_sc as plsc

K = 16
B = 8
N = 2048
C = 64
C2 = 2 * C
RT = 256
NT = N // RT
BN = B * N
ROWS = BN * K
HROWS = ROWS // 2
M_ELEMS = float(ROWS)
NEG = -3.0e38
EPS = 1e-5

NC = 2
NS = 16
NW = NC * NS
ROWS_W = ROWS // NW
SUB = 128
NSUB = ROWS_W // SUB
CHROWS = 512
NCH = ROWS_W // CHROWS
SUB_CH = CHROWS // SUB


def _knn_pre_body(ptsc_ref, ptsr_ref, ft_ref, wdt_ref, wbt_ref,
                  idx_ref, a_ref, g_ref):
    b = pl.program_id(0)
    ptsc = ptsc_ref[0]
    ptsr = ptsr_ref[0]
    xx_c = jnp.sum(ptsc * ptsc, axis=0, keepdims=True)
    xx_r = jnp.sum(ptsr * ptsr, axis=1, keepdims=True)
    ptscb = ptsc.astype(jnp.bfloat16).astype(jnp.float32)
    ptsrb = ptsr.astype(jnp.bfloat16).astype(jnp.float32)
    acc = (ptsrb[:, 0:1] * ptscb[0:1, :]
           + ptsrb[:, 1:2] * ptscb[1:2, :]
           + ptsrb[:, 2:3] * ptscb[2:3, :])
    d = (-xx_r - (-2.0 * acc)) - xx_c
    colv = lax.broadcasted_iota(jnp.int32, (1, N), 1)
    hi = (colv // 16).astype(jnp.float32)
    lo = (colv % 16).astype(jnp.float32)
    hilo = jnp.concatenate([hi, lo], axis=0)
    picked = []
    for j in range(K + 1):
        m = jnp.max(d, axis=1, keepdims=True)
        eq = d == m
        if j > 0:
            eqf = jnp.where(eq, 1.0, 0.0)
            hl = jax.lax.dot_general(
                eqf, hilo, (((1,), (1,)), ((), ())),
                preferred_element_type=jnp.float32)
            picked.append(hl[:, 0:1] * 16.0 + hl[:, 1:2])
        if j < K:
            d = jnp.where(eq, NEG, d)
    sel = jnp.minimum(jnp.concatenate(picked, axis=1), float(N - 1))
    idx_ref[0] = sel.astype(jnp.int32) + b * N
    ft = ft_ref[0]
    a_ref[0] = jnp.dot(ft, wdt_ref[...], precision=lax.Precision.HIGHEST,
                       preferred_element_type=jnp.float32)
    g_ref[0] = jnp.dot(ft, wbt_ref[...], precision=lax.Precision.HIGHEST,
                       preferred_element_type=jnp.float32)


_knn_pre = pl.pallas_call(
    _knn_pre_body,
    grid=(B, NT),
    in_specs=[
        pl.BlockSpec((1, 3, N), lambda b, t: (b, 0, 0)),
        pl.BlockSpec((1, RT, 3), lambda b, t: (b, t, 0)),
        pl.BlockSpec((1, RT, C), lambda b, t: (b, t, 0)),
        pl.BlockSpec((C, C), lambda b, t: (0, 0)),
        pl.BlockSpec((C, C), lambda b, t: (0, 0)),
    ],
    out_specs=[
        pl.BlockSpec((1, RT, K), lambda b, t: (b, t, 0)),
        pl.BlockSpec((1, RT, C), lambda b, t: (b, t, 0)),
        pl.BlockSpec((1, RT, C), lambda b, t: (b, t, 0)),
    ],
    out_shape=[
        jax.ShapeDtypeStruct((B, N, K), jnp.int32),
        jax.ShapeDtypeStruct((B, N, C), jnp.float32),
        jax.ShapeDtypeStruct((B, N, C), jnp.float32),
    ],
)


def _sc_gather_body(table_hbm, idx_hbm, out_hbm, idx_v, rows_a, rows_b, sem,
                    semo):
    wid = lax.axis_index("s") * NC + lax.axis_index("c")
    pltpu.sync_copy(idx_hbm.at[wid], idx_v)
    bufs = (rows_a, rows_b)
    pending = [None, None]
    for ch in range(NCH):
        buf = bufs[ch % 2]
        if pending[ch % 2] is not None:
            pending[ch % 2].wait()
        descs = []
        for j in range(SUB_CH):
            descs.append(pltpu.async_copy(
                table_hbm.at[idx_v.at[ch * SUB_CH + j]],
                buf.at[pl.ds(j * SUB, SUB)], sem))
        for dsc in descs:
            dsc.wait()
        pending[ch % 2] = pltpu.async_copy(
            buf, out_hbm.at[pl.ds(wid * ROWS_W + ch * CHROWS, CHROWS)], semo)
    pending[0].wait()
    pending[1].wait()


@functools.cache
def _make_sc_gather():
    return functools.partial(
        pl.kernel,
        out_type=jax.ShapeDtypeStruct((ROWS, C), jnp.float32),
        mesh=plsc.VectorSubcoreMesh(core_axis_name="c", subcore_axis_name="s",
                                    num_cores=NC, num_subcores=NS),
        scratch_types=[
            pltpu.VMEM((NSUB, SUB), jnp.int32),
            pltpu.VMEM((CHROWS, C), jnp.float32),
            pltpu.VMEM((CHROWS, C), jnp.float32),
            pltpu.SemaphoreType.DMA,
            pltpu.SemaphoreType.DMA,
        ],
        compiler_params=pltpu.CompilerParams(use_tc_tiling_on_sc=False),
    )(_sc_gather_body)


HROWB = 4096
NODB = HROWB // 8
NSTEP = HROWS // HROWB


def _bn_coeffs(st_ref, g_ref, b_ref):
    s = st_ref[0:1, :C] + st_ref[0:1, C:]
    ss = st_ref[1:2, :C] + st_ref[1:2, C:]
    mean = s / M_ELEMS
    var = ss / M_ELEMS - mean * mean
    scale = g_ref[...] * lax.rsqrt(var + EPS)
    shift = b_ref[...] - mean * scale
    scale2 = jnp.concatenate([scale, scale], axis=1)
    shift2 = jnp.concatenate([shift, shift], axis=1)
    return scale2, shift2


def _acc_stats(st_ref, x):
    i = pl.program_id(0)

    @pl.when(i == 0)
    def _():
        st_ref[...] = jnp.zeros_like(st_ref)

    s = jnp.sum(x, axis=0, keepdims=True)
    ss = jnp.sum(x * x, axis=0, keepdims=True)
    st_ref[0:1, :] = st_ref[0:1, :] + s
    st_ref[1:2, :] = st_ref[1:2, :] + ss


def _x1_block(gg_ref, a_ref):
    a2 = jnp.concatenate([a_ref[...], a_ref[...]], axis=1)
    return (gg_ref[...].reshape(NODB, 8, C2)
            + a2[:, None, :]).reshape(HROWB, C2)


def _stats1_body(gg_ref, a_ref, st_ref):
    _acc_stats(st_ref, _x1_block(gg_ref, a_ref))


_stats1 = pl.pallas_call(
    _stats1_body,
    grid=(NSTEP,),
    in_specs=[
        pl.BlockSpec((HROWB, C2), lambda i: (i, 0)),
        pl.BlockSpec((NODB, C), lambda i: (i, 0)),
    ],
    out_specs=pl.BlockSpec((8, C2), lambda i: (0, 0)),
    out_shape=jax.ShapeDtypeStruct((8, C2), jnp.float32),
)


def _p5_body(gg_ref, a_ref, st_ref, w_ref, g_ref, b_ref, x2_ref, st2_ref):
    scale, shift = _bn_coeffs(st_ref, g_ref, b_ref)
    x1 = _x1_block(gg_ref, a_ref)
    y = jnp.maximum(x1 * scale + shift, 0.0).astype(jnp.bfloat16)
    x2 = jnp.dot(y, w_ref[...], preferred_element_type=jnp.float32)
    x2_ref[...] = x2
    _acc_stats(st2_ref, x2)


_p5 = pl.pallas_call(
    _p5_body,
    grid=(NSTEP,),
    in_specs=[
        pl.BlockSpec((HROWB, C2), lambda i: (i, 0)),
        pl.BlockSpec((NODB, C), lambda i: (i, 0)),
        pl.BlockSpec((8, C2), lambda i: (0, 0)),
        pl.BlockSpec((C2, C2), lambda i: (0, 0)),
        pl.BlockSpec((1, C), lambda i: (0, 0)),
        pl.BlockSpec((1, C), lambda i: (0, 0)),
    ],
    out_specs=[
        pl.BlockSpec((HROWB, C2), lambda i: (i, 0)),
        pl.BlockSpec((8, C2), lambda i: (0, 0)),
    ],
    out_shape=[
        jax.ShapeDtypeStruct((HROWS, C2), jnp.float32),
        jax.ShapeDtypeStruct((8, C2), jnp.float32),
    ],
)


def _p6_body(x2_ref, st_ref, w_ref, g_ref, b_ref, x3_ref, st3_ref):
    scale, shift = _bn_coeffs(st_ref, g_ref, b_ref)
    y = jnp.maximum(x2_ref[...] * scale + shift, 0.0).astype(jnp.bfloat16)
    x3 = jnp.dot(y, w_ref[...], preferred_element_type=jnp.float32)
    x3_ref[...] = x3
    _acc_stats(st3_ref, x3)


_p6 = pl.pallas_call(
    _p6_body,
    grid=(NSTEP,),
    in_specs=[
        pl.BlockSpec((HROWB, C2), lambda i: (i, 0)),
        pl.BlockSpec((8, C2), lambda i: (0, 0)),
        pl.BlockSpec((C2, C2), lambda i: (0, 0)),
        pl.BlockSpec((1, C), lambda i: (0, 0)),
        pl.BlockSpec((1, C), lambda i: (0, 0)),
    ],
    out_specs=[
        pl.BlockSpec((HROWB, C2), lambda i: (i, 0)),
        pl.BlockSpec((8, C2), lambda i: (0, 0)),
    ],
    out_shape=[
        jax.ShapeDtypeStruct((HROWS, C2), jnp.float32),
        jax.ShapeDtypeStruct((8, C2), jnp.float32),
    ],
)


def _p7_body(x3_ref, st_ref, ft_ref, g_ref, b_ref, o_ref):
    scale, shift = _bn_coeffs(st_ref, g_ref, b_ref)
    y = jnp.maximum(x3_ref[...] * scale + shift, 0.0)
    acc = jnp.sum(y.reshape(NODB, 8, C2), axis=1)
    fts = (acc[:, :C] + acc[:, C:]) * (1.0 / K)
    o_ref[...] = jnp.maximum(ft_ref[...] + fts, 0.0)


_p7 = pl.pallas_call(
    _p7_body,
    grid=(NSTEP,),
    in_specs=[
        pl.BlockSpec((HROWB, C2), lambda i: (i, 0)),
        pl.BlockSpec((8, C2), lambda i: (0, 0)),
        pl.BlockSpec((NODB, C), lambda i: (i, 0)),
        pl.BlockSpec((1, C), lambda i: (0, 0)),
        pl.BlockSpec((1, C), lambda i: (0, 0)),
    ],
    out_specs=pl.BlockSpec((NODB, C), lambda i: (i, 0)),
    out_shape=jax.ShapeDtypeStruct((BN, C), jnp.float32),
)


def _dup_blockdiag(w):
    wt = jnp.transpose(w).astype(jnp.bfloat16)
    z = jnp.zeros((C, C), jnp.bfloat16)
    return jnp.concatenate(
        [jnp.concatenate([wt, z], axis=1),
         jnp.concatenate([z, wt], axis=1)], axis=0)


def kernel(points, features, W0, W1, W2, g0, b0, g1, b1, g2, b2):
    ft = jnp.transpose(features, (0, 2, 1))
    pt = jnp.transpose(points, (0, 2, 1))
    wdt = jnp.transpose(W0[:, :C] - W0[:, C:])
    wbt = jnp.transpose(W0[:, C:])
    idx, a_nodes, g_nodes = _knn_pre(points, pt, ft, wdt, wbt)
    gg = _make_sc_gather()(g_nodes.reshape(BN, C), idx.reshape(NW, NSUB, SUB))
    gg2 = gg.reshape(HROWS, C2)
    a2 = a_nodes.reshape(BN, C)
    st1 = _stats1(gg2, a2)
    x2, st2 = _p5(gg2, a2, st1, _dup_blockdiag(W1),
                  g0.reshape(1, C), b0.reshape(1, C))
    x3, st3 = _p6(x2, st2, _dup_blockdiag(W2),
                  g1.reshape(1, C), b1.reshape(1, C))
    out = _p7(x3, st3, ft.reshape(BN, C),
              g2.reshape(1, C), b2.reshape(1, C))
    return jnp.transpose(out.reshape(B, N, C), (0, 2, 1))


# speedup vs baseline: 1.0221x; 1.0095x over previous
import functools

import jax
import jax.numpy as jnp
from jax import lax
from jax.experimental import pallas as pl
from jax.experimental.pallas import tpu as pltpu
from jax.experimental.pallas import tpu_sc as plsc

K = 16
B = 8
N = 2048
C = 64
C2 = 2 * C
RT = 256
NT = N // RT
BN = B * N
ROWS = BN * K
HROWS = ROWS // 2
M_ELEMS = float(ROWS)
NEG = -3.0e38
EPS = 1e-5

NC = 2
NS = 16
NW = NC * NS
ROWS_W = ROWS // NW
SUB = 128
NSUB = ROWS_W // SUB
CHROWS = 512
NCH = ROWS_W // CHROWS
SUB_CH = CHROWS // SUB


def _knn_pre_body(ptsc_ref, ptsr_ref, ft_ref, wdt_ref, wbt_ref,
                  idx_ref, a_ref, g_ref):
    b = pl.program_id(0)
    ptsc = ptsc_ref[0]
    ptsr = ptsr_ref[0]
    xx_c = jnp.sum(ptsc * ptsc, axis=0, keepdims=True)
    xx_r = jnp.sum(ptsr * ptsr, axis=1, keepdims=True)
    ptscb = ptsc.astype(jnp.bfloat16).astype(jnp.float32)
    ptsrb = ptsr.astype(jnp.bfloat16).astype(jnp.float32)
    acc = (ptsrb[:, 0:1] * ptscb[0:1, :]
           + ptsrb[:, 1:2] * ptscb[1:2, :]
           + ptsrb[:, 2:3] * ptscb[2:3, :])
    d = (-xx_r - (-2.0 * acc)) - xx_c
    colv = lax.broadcasted_iota(jnp.int32, (1, N), 1)
    hi = (colv // 16).astype(jnp.float32)
    lo = (colv % 16).astype(jnp.float32)
    hilo = jnp.concatenate([hi, lo], axis=0)
    picked = []
    for j in range(K + 1):
        m = jnp.max(d, axis=1, keepdims=True)
        eq = d == m
        if j > 0:
            eqf = jnp.where(eq, 1.0, 0.0)
            hl = jax.lax.dot_general(
                eqf, hilo, (((1,), (1,)), ((), ())),
                preferred_element_type=jnp.float32)
            picked.append(hl[:, 0:1] * 16.0 + hl[:, 1:2])
        if j < K:
            d = jnp.where(eq, NEG, d)
    sel = jnp.minimum(jnp.concatenate(picked, axis=1), float(N - 1))
    idx_ref[0] = sel.astype(jnp.int32) + b * N
    ft = ft_ref[0]
    a_ref[0] = jnp.dot(ft, wdt_ref[...], precision=lax.Precision.HIGHEST,
                       preferred_element_type=jnp.float32)
    g_ref[0] = jnp.dot(ft, wbt_ref[...], precision=lax.Precision.HIGHEST,
                       preferred_element_type=jnp.float32)


_knn_pre = pl.pallas_call(
    _knn_pre_body,
    grid=(B, NT),
    in_specs=[
        pl.BlockSpec((1, 3, N), lambda b, t: (b, 0, 0)),
        pl.BlockSpec((1, RT, 3), lambda b, t: (b, t, 0)),
        pl.BlockSpec((1, RT, C), lambda b, t: (b, t, 0)),
        pl.BlockSpec((C, C), lambda b, t: (0, 0)),
        pl.BlockSpec((C, C), lambda b, t: (0, 0)),
    ],
    out_specs=[
        pl.BlockSpec((1, RT, K), lambda b, t: (b, t, 0)),
        pl.BlockSpec((1, RT, C), lambda b, t: (b, t, 0)),
        pl.BlockSpec((1, RT, C), lambda b, t: (b, t, 0)),
    ],
    out_shape=[
        jax.ShapeDtypeStruct((B, N, K), jnp.int32),
        jax.ShapeDtypeStruct((B, N, C), jnp.float32),
        jax.ShapeDtypeStruct((B, N, C), jnp.float32),
    ],
)


def _sc_gather_body(table_hbm, idx_hbm, out_hbm, idx_v, rows_a, rows_b, sem,
                    semo):
    wid = lax.axis_index("s") * NC + lax.axis_index("c")
    pltpu.sync_copy(idx_hbm.at[wid], idx_v)
    bufs = (rows_a, rows_b)
    pending = [None, None]
    for ch in range(NCH):
        buf = bufs[ch % 2]
        if pending[ch % 2] is not None:
            pending[ch % 2].wait()
        descs = []
        for j in range(SUB_CH):
            descs.append(pltpu.async_copy(
                table_hbm.at[idx_v.at[ch * SUB_CH + j]],
                buf.at[pl.ds(j * SUB, SUB)], sem))
        for dsc in descs:
            dsc.wait()
        pending[ch % 2] = pltpu.async_copy(
            buf, out_hbm.at[pl.ds(wid * ROWS_W + ch * CHROWS, CHROWS)], semo)
    pending[0].wait()
    pending[1].wait()


@functools.cache
def _make_sc_gather():
    return functools.partial(
        pl.kernel,
        out_type=jax.ShapeDtypeStruct((ROWS, C), jnp.float32),
        mesh=plsc.VectorSubcoreMesh(core_axis_name="c", subcore_axis_name="s",
                                    num_cores=NC, num_subcores=NS),
        scratch_types=[
            pltpu.VMEM((NSUB, SUB), jnp.int32),
            pltpu.VMEM((CHROWS, C), jnp.float32),
            pltpu.VMEM((CHROWS, C), jnp.float32),
            pltpu.SemaphoreType.DMA,
            pltpu.SemaphoreType.DMA,
        ],
        compiler_params=pltpu.CompilerParams(use_tc_tiling_on_sc=False),
    )(_sc_gather_body)


HROWB = 4096
NODB = HROWB // 8
NSTEP = HROWS // HROWB


def _bn_coeffs(st_ref, g_ref, b_ref):
    s = st_ref[0:1, :C] + st_ref[0:1, C:]
    ss = st_ref[1:2, :C] + st_ref[1:2, C:]
    mean = s / M_ELEMS
    var = ss / M_ELEMS - mean * mean
    scale = g_ref[...] * lax.rsqrt(var + EPS)
    shift = b_ref[...] - mean * scale
    scale2 = jnp.concatenate([scale, scale], axis=1)
    shift2 = jnp.concatenate([shift, shift], axis=1)
    return scale2, shift2


def _acc_stats(st_ref, x):
    i = pl.program_id(0)

    @pl.when(i == 0)
    def _():
        st_ref[...] = jnp.zeros_like(st_ref)

    s = jnp.sum(x, axis=0, keepdims=True)
    ss = jnp.sum(x * x, axis=0, keepdims=True)
    st_ref[0:1, :] = st_ref[0:1, :] + s
    st_ref[1:2, :] = st_ref[1:2, :] + ss


def _x1_block(gg_ref, a_ref):
    a2 = jnp.concatenate([a_ref[...], a_ref[...]], axis=1)
    return (gg_ref[...].reshape(NODB, 8, C2)
            + a2[:, None, :]).reshape(HROWB, C2)


def _stats1_body(gg_ref, a_ref, st_ref):
    _acc_stats(st_ref, _x1_block(gg_ref, a_ref))


_stats1 = pl.pallas_call(
    _stats1_body,
    grid=(NSTEP,),
    in_specs=[
        pl.BlockSpec((HROWB, C2), lambda i: (i, 0)),
        pl.BlockSpec((NODB, C), lambda i: (i, 0)),
    ],
    out_specs=pl.BlockSpec((8, C2), lambda i: (0, 0)),
    out_shape=jax.ShapeDtypeStruct((8, C2), jnp.float32),
)


def _p5_body(gg_ref, a_ref, st_ref, w_ref, g_ref, b_ref, x2_ref, st2_ref):
    scale, shift = _bn_coeffs(st_ref, g_ref, b_ref)
    x1 = _x1_block(gg_ref, a_ref)
    y = jnp.maximum(x1 * scale + shift, 0.0).astype(jnp.bfloat16)
    x2 = jnp.dot(y, w_ref[...], preferred_element_type=jnp.float32)
    x2_ref[...] = x2
    _acc_stats(st2_ref, x2)


_p5 = pl.pallas_call(
    _p5_body,
    grid=(NSTEP,),
    in_specs=[
        pl.BlockSpec((HROWB, C2), lambda i: (i, 0)),
        pl.BlockSpec((NODB, C), lambda i: (i, 0)),
        pl.BlockSpec((8, C2), lambda i: (0, 0)),
        pl.BlockSpec((C2, C2), lambda i: (0, 0)),
        pl.BlockSpec((1, C), lambda i: (0, 0)),
        pl.BlockSpec((1, C), lambda i: (0, 0)),
    ],
    out_specs=[
        pl.BlockSpec((HROWB, C2), lambda i: (i, 0)),
        pl.BlockSpec((8, C2), lambda i: (0, 0)),
    ],
    out_shape=[
        jax.ShapeDtypeStruct((HROWS, C2), jnp.float32),
        jax.ShapeDtypeStruct((8, C2), jnp.float32),
    ],
)


def _p6_body(x2_ref, st_ref, w_ref, g_ref, b_ref, x3_ref, st3_ref):
    scale, shift = _bn_coeffs(st_ref, g_ref, b_ref)
    y = jnp.maximum(x2_ref[...] * scale + shift, 0.0).astype(jnp.bfloat16)
    x3 = jnp.dot(y, w_ref[...], preferred_element_type=jnp.float32)
    x3_ref[...] = x3
    _acc_stats(st3_ref, x3)


_p6 = pl.pallas_call(
    _p6_body,
    grid=(NSTEP,),
    in_specs=[
        pl.BlockSpec((HROWB, C2), lambda i: (i, 0)),
        pl.BlockSpec((8, C2), lambda i: (0, 0)),
        pl.BlockSpec((C2, C2), lambda i: (0, 0)),
        pl.BlockSpec((1, C), lambda i: (0, 0)),
        pl.BlockSpec((1, C), lambda i: (0, 0)),
    ],
    out_specs=[
        pl.BlockSpec((HROWB, C2), lambda i: (i, 0)),
        pl.BlockSpec((8, C2), lambda i: (0, 0)),
    ],
    out_shape=[
        jax.ShapeDtypeStruct((HROWS, C2), jnp.float32),
        jax.ShapeDtypeStruct((8, C2), jnp.float32),
    ],
)


def _p7_body(x3_ref, st_ref, ft_ref, g_ref, b_ref, o_ref):
    scale, shift = _bn_coeffs(st_ref, g_ref, b_ref)
    y = jnp.maximum(x3_ref[...] * scale + shift, 0.0)
    acc = jnp.sum(y.reshape(NODB, 8, C2), axis=1)
    fts = (acc[:, :C] + acc[:, C:]) * (1.0 / K)
    res = jnp.maximum(ft_ref[...] + fts, 0.0)
    o_ref[0] = jnp.transpose(res)


_p7 = pl.pallas_call(
    _p7_body,
    grid=(NSTEP,),
    in_specs=[
        pl.BlockSpec((HROWB, C2), lambda i: (i, 0)),
        pl.BlockSpec((8, C2), lambda i: (0, 0)),
        pl.BlockSpec((NODB, C), lambda i: (i, 0)),
        pl.BlockSpec((1, C), lambda i: (0, 0)),
        pl.BlockSpec((1, C), lambda i: (0, 0)),
    ],
    out_specs=pl.BlockSpec((1, C, NODB), lambda i: (i // 4, 0, i % 4)),
    out_shape=jax.ShapeDtypeStruct((B, C, N), jnp.float32),
)


def _dup_blockdiag(w):
    wt = jnp.transpose(w).astype(jnp.bfloat16)
    z = jnp.zeros((C, C), jnp.bfloat16)
    return jnp.concatenate(
        [jnp.concatenate([wt, z], axis=1),
         jnp.concatenate([z, wt], axis=1)], axis=0)


def kernel(points, features, W0, W1, W2, g0, b0, g1, b1, g2, b2):
    ft = jnp.transpose(features, (0, 2, 1))
    pt = jnp.transpose(points, (0, 2, 1))
    wdt = jnp.transpose(W0[:, :C] - W0[:, C:])
    wbt = jnp.transpose(W0[:, C:])
    idx, a_nodes, g_nodes = _knn_pre(points, pt, ft, wdt, wbt)
    gg = _make_sc_gather()(g_nodes.reshape(BN, C), idx.reshape(NW, NSUB, SUB))
    gg2 = gg.reshape(HROWS, C2)
    a2 = a_nodes.reshape(BN, C)
    st1 = _stats1(gg2, a2)
    x2, st2 = _p5(gg2, a2, st1, _dup_blockdiag(W1),
                  g0.reshape(1, C), b0.reshape(1, C))
    x3, st3 = _p6(x2, st2, _dup_blockdiag(W2),
                  g1.reshape(1, C), b1.reshape(1, C))
    return _p7(x3, st3, ft.reshape(BN, C),
               g2.reshape(1, C), b2.reshape(1, C))
